# baseline (speedup 1.0000x reference)
import functools

import jax
import jax.numpy as jnp
from jax import lax
from jax.experimental import pallas as pl
from jax.experimental.pallas import tpu as pltpu
from jax.experimental.pallas import tpu_sc as plsc

N = 10000
E = 320000
D = 128
H = 128

NC = 2
NS = 16
NW = NC * NS

NP = 10112
ROWS_PER_TILE = NP // NS
E_PER_W = E // NW
E_PER_W_PAD = 10368
CH = 128
CHUNKS = E_PER_W_PAD // CH
SCH = 96
SCHUNKS = E_PER_W_PAD // SCH

H_STRIPE = 624
H_STRIPE_LAST = N - 15 * H_STRIPE

_sc_mesh = plsc.VectorSubcoreMesh(core_axis_name="c", subcore_axis_name="s")


@functools.partial(
    pl.kernel,
    out_type=jax.ShapeDtypeStruct((NW, E_PER_W_PAD, D), jnp.float32),
    mesh=_sc_mesh,
    scratch_types=[
        pltpu.VMEM((E_PER_W_PAD,), jnp.int32),
        pltpu.VMEM((CH, D), jnp.float32),
        pltpu.VMEM((CH, D), jnp.float32),
        pltpu.VMEM_SHARED((N, D), jnp.float32),
        pltpu.SemaphoreType.DMA,
        pltpu.SemaphoreType.DMA,
        pltpu.SemaphoreType.DMA,
        pltpu.SemaphoreType.DMA,
    ],
)
def _sc_materialize(h_hbm, src_hbm, msg_hbm,
                    src_v, rows0_v, rows1_v, h_spm,
                    gsem0, gsem1, wsem0, wsem1):
    c = lax.axis_index("c")
    s = lax.axis_index("s")
    wid = s * NC + c

    @pl.when(s < 15)
    def _():
        r0 = pl.multiple_of(s * H_STRIPE, 8)
        pltpu.sync_copy(h_hbm.at[pl.ds(r0, H_STRIPE)],
                        h_spm.at[pl.ds(r0, H_STRIPE)])

    @pl.when(s == 15)
    def _():
        pltpu.sync_copy(h_hbm.at[pl.ds(15 * H_STRIPE, H_STRIPE_LAST)],
                        h_spm.at[pl.ds(15 * H_STRIPE, H_STRIPE_LAST)])

    pltpu.sync_copy(src_hbm.at[wid], src_v)
    plsc.subcore_barrier()

    def _start_gather(j, buf, sem):
        return pltpu.async_copy(h_spm.at[src_v.at[pl.ds(j * CH, CH)]],
                                buf, sem)

    def _start_write(j, buf, sem):
        return pltpu.async_copy(
            buf, msg_hbm.at[wid, pl.ds(pl.multiple_of(j * CH, CH), CH)], sem)

    def _wait(buf, sem):
        pltpu.make_async_copy(h_hbm.at[pl.ds(0, CH)], buf, sem).wait()

    _start_gather(0, rows0_v, gsem0)
    _start_gather(1, rows1_v, gsem1)

    def body(i, carry):
        j = pl.multiple_of(i * 2, 2)
        _wait(rows0_v, gsem0)
        _start_write(j, rows0_v, wsem0)
        _wait(rows1_v, gsem1)
        _start_write(j + 1, rows1_v, wsem1)

        @pl.when(j + 2 < CHUNKS)
        def _():
            _wait(rows0_v, wsem0)
            _start_gather(j + 2, rows0_v, gsem0)

        @pl.when(j + 3 < CHUNKS)
        def _():
            _wait(rows1_v, wsem1)
            _start_gather(j + 3, rows1_v, gsem1)

        return carry

    lax.fori_loop(0, CHUNKS // 2, body, 0, unroll=False)
    _wait(rows0_v, gsem0)
    _start_write(CHUNKS - 1, rows0_v, wsem0)
    _wait(rows0_v, wsem0)
    _wait(rows1_v, wsem1)


@functools.partial(
    pl.kernel,
    out_type=jax.ShapeDtypeStruct((NC, NP, D), jnp.float32),
    mesh=_sc_mesh,
    scratch_types=[
        pltpu.VMEM((6, SCH), jnp.int32),
        pltpu.VMEM((SCH, D), jnp.float32),
        pltpu.VMEM((SCH, D), jnp.float32),
        pltpu.VMEM((SCH, D), jnp.float32),
        pltpu.VMEM((SCH, D), jnp.float32),
        pltpu.VMEM_SHARED((NP, D), jnp.float32),
        pltpu.SemaphoreType.DMA,
        pltpu.SemaphoreType.DMA,
        pltpu.SemaphoreType.DMA,
        pltpu.SemaphoreType.DMA,
        pltpu.SemaphoreType.DMA,
        pltpu.SemaphoreType.DMA,
        pltpu.SemaphoreType.DMA,
        pltpu.SemaphoreType.DMA,
        pltpu.SemaphoreType.DMA,
        pltpu.SemaphoreType.DMA,
        pltpu.SemaphoreType.DMA,
    ],
)
def _sc_scatter(msg_hbm, dst_hbm, zeros_hbm, out_hbm,
                ring_v, rows0_v, rows1_v, rows2_v, rows3_v, acc,
                rsem0, rsem1, rsem2, rsem3,
                ssem0, ssem1, ssem2, ssem3,
                isem0, isem1, isem2):
    c = lax.axis_index("c")
    s = lax.axis_index("s")
    wid = s * NC + c
    r0 = pl.multiple_of(s * ROWS_PER_TILE, ROWS_PER_TILE)
    pltpu.sync_copy(zeros_hbm.at[pl.ds(r0, ROWS_PER_TILE)],
                    acc.at[pl.ds(r0, ROWS_PER_TILE)])
    plsc.subcore_barrier()

    rows = (rows0_v, rows1_v, rows2_v, rows3_v)
    rsem = (rsem0, rsem1, rsem2, rsem3)
    ssem = (ssem0, ssem1, ssem2, ssem3)
    isem = (isem0, isem1, isem2)

    def _start_read(j, b, sem):
        return pltpu.async_copy(
            msg_hbm.at[wid, pl.ds(pl.multiple_of(j * SCH, SCH), SCH)],
            rows[b], sem)

    def _start_idx(j, slot, sem):
        return pltpu.async_copy(dst_hbm.at[wid, j], ring_v.at[slot], sem)

    def _wait_rows(b, sem):
        pltpu.make_async_copy(msg_hbm.at[0, pl.ds(0, SCH)], rows[b],
                              sem).wait()

    def _wait_idx(slot, sem):
        pltpu.make_async_copy(dst_hbm.at[0, 0], ring_v.at[slot], sem).wait()

    def _chunk(j, b, i3, slot6, first12):
        _wait_rows(b, rsem[b])
        _wait_idx(slot6, isem[i3])
        pltpu.async_copy(rows[b], acc.at[ring_v.at[slot6]], ssem[b],
                         add=True)
        nb = (b + 2) % 4
        if first12:
            if isinstance(j, int) and j < 2:
                _start_read(j + 2, nb, rsem[nb])
            else:
                _wait_rows(nb, ssem[nb])
                _start_read(j + 2, nb, rsem[nb])
        else:
            _wait_rows(nb, ssem[nb])
            _start_read(j + 2, nb, rsem[nb])
        _start_idx(j + 3, (slot6 + 3) % 6, isem[i3])

    _start_read(0, 0, rsem[0])
    _start_read(1, 1, rsem[1])
    _start_idx(0, 0, isem[0])
    _start_idx(1, 1, isem[1])
    _start_idx(2, 2, isem[2])
    for j in range(12):
        _chunk(j, j % 4, j % 3, j % 6, True)

    def body(i, carry):
        j0 = pl.multiple_of(i * 12, 12)
        for k in range(12):
            _chunk(j0 + k, k % 4, k % 3, k % 6, False)
        return carry

    lax.fori_loop(1, 8, body, 0, unroll=False)
    for j in range(96, 108):
        b, i3, slot6 = j % 4, j % 3, j % 6
        _wait_rows(b, rsem[b])
        _wait_idx(slot6, isem[i3])
        pltpu.async_copy(rows[b], acc.at[ring_v.at[slot6]], ssem[b],
                         add=True)
        nb = (b + 2) % 4
        if j + 2 < 108:
            _wait_rows(nb, ssem[nb])
            _start_read(j + 2, nb, rsem[nb])
        if j + 3 < 108:
            _start_idx(j + 3, (slot6 + 3) % 6, isem[i3])
    _wait_rows(0, ssem[0])
    _wait_rows(1, ssem[1])
    _wait_rows(2, ssem[2])
    _wait_rows(3, ssem[3])
    plsc.subcore_barrier()
    pltpu.sync_copy(acc.at[pl.ds(r0, ROWS_PER_TILE)],
                    out_hbm.at[c, pl.ds(r0, ROWS_PER_TILE)])


def _sc_segment_sum(h, src, dst, zeros):
    msg = _sc_materialize(h, src)
    return _sc_scatter(msg, dst, zeros)


def _tc_linear_body(x_ref, w_ref, b_ref, o_ref):
    o_ref[...] = (jnp.dot(x_ref[...], w_ref[...],
                          preferred_element_type=jnp.float32) + b_ref[...])


def _tc_linear(x, W, b):
    nb = 10
    return pl.pallas_call(
        _tc_linear_body,
        grid=(nb,),
        in_specs=[
            pl.BlockSpec((N // nb, D), lambda i: (i, 0)),
            pl.BlockSpec((D, H), lambda i: (0, 0)),
            pl.BlockSpec((1, H), lambda i: (0, 0)),
        ],
        out_specs=pl.BlockSpec((N // nb, H), lambda i: (i, 0)),
        out_shape=jax.ShapeDtypeStruct((N, H), jnp.float32),
    )(x, W, b.reshape(1, H))


def _tc_tail_body(relu_out, h_ref, p_ref, wa_ref, ba_ref, g_ref, be_ref,
                  wb_ref, bb_ref, o_ref):
    out = h_ref[...] + p_ref[0, :N] + p_ref[1, :N]
    t = (jnp.dot(out, wa_ref[...], preferred_element_type=jnp.float32)
         + ba_ref[...])
    m = jnp.mean(t, axis=0, keepdims=True)
    v = jnp.mean((t - m) ** 2, axis=0, keepdims=True)
    t = (t - m) * lax.rsqrt(v + 1e-5) * g_ref[...] + be_ref[...]
    t = jnp.maximum(t, 0.0)
    l = (jnp.dot(t, wb_ref[...], preferred_element_type=jnp.float32)
         + bb_ref[...])
    cm = jnp.mean(l, axis=0, keepdims=True)
    rn = jnp.sqrt(1e-6 + jnp.sum(l * l, axis=1, keepdims=True))
    res = 20.0 * l / rn - cm
    if relu_out:
        res = jnp.maximum(res, 0.0)
    o_ref[...] = res


def _tc_tail(relu_out, h, p, Wa, ba, g, be, Wb, bb):
    return pl.pallas_call(
        functools.partial(_tc_tail_body, relu_out),
        out_shape=jax.ShapeDtypeStruct((N, H), jnp.float32),
    )(h, p, Wa, ba.reshape(1, H), g.reshape(1, H), be.reshape(1, H),
      Wb, bb.reshape(1, H))


def kernel(x, edge_index, W0, b0, W1a, b1a, g1, be1, W1b, b1b,
           W2a, b2a, g2, be2, W2b, b2b):
    pad = E_PER_W_PAD - E_PER_W
    src = jnp.pad(edge_index[0].reshape(NW, E_PER_W), ((0, 0), (0, pad)),
                  constant_values=0)
    dst = jnp.pad(edge_index[1].reshape(NW, E_PER_W), ((0, 0), (0, pad)),
                  constant_values=N).reshape(NW, SCHUNKS, SCH)
    zeros = jnp.zeros((NP, D), jnp.float32)

    h0 = _tc_linear(x, W0, b0)
    p1 = _sc_segment_sum(h0, src, dst, zeros)
    l1 = _tc_tail(True, h0, p1, W1a, b1a, g1, be1, W1b, b1b)
    p2 = _sc_segment_sum(l1, src, dst, zeros)
    l2 = _tc_tail(False, l1, p2, W2a, b2a, g2, be2, W2b, b2b)
    return l2


# speedup vs baseline: 1.0079x; 1.0079x over previous
import functools

import jax
import jax.numpy as jnp
from jax import lax
from jax.experimental import pallas as pl
from jax.experimental.pallas import tpu as pltpu
from jax.experimental.pallas import tpu_sc as plsc

N = 10000
E = 320000
D = 128
H = 128

NC = 2
NS = 16
NW = NC * NS

NP = 10112
ROWS_PER_TILE = NP // NS
E_PER_W = E // NW
E_PER_W_PAD = 10368
CH = 128
CHUNKS = E_PER_W_PAD // CH
SCH = 96
SCHUNKS = E_PER_W_PAD // SCH

H_STRIPE = 624
H_STRIPE_LAST = N - 15 * H_STRIPE

_sc_mesh = plsc.VectorSubcoreMesh(core_axis_name="c", subcore_axis_name="s")


@functools.partial(
    pl.kernel,
    out_type=jax.ShapeDtypeStruct((NW, E_PER_W_PAD, D), jnp.float32),
    mesh=_sc_mesh,
    scratch_types=[
        pltpu.VMEM((E_PER_W_PAD,), jnp.int32),
        pltpu.VMEM((CH, D), jnp.float32),
        pltpu.VMEM((CH, D), jnp.float32),
        pltpu.VMEM_SHARED((N, D), jnp.float32),
        pltpu.SemaphoreType.DMA,
        pltpu.SemaphoreType.DMA,
        pltpu.SemaphoreType.DMA,
        pltpu.SemaphoreType.DMA,
    ],
)
def _sc_materialize(h_hbm, src_hbm, msg_hbm,
                    src_v, rows0_v, rows1_v, h_spm,
                    gsem0, gsem1, wsem0, wsem1):
    c = lax.axis_index("c")
    s = lax.axis_index("s")
    wid = s * NC + c

    @pl.when(s < 15)
    def _():
        r0 = pl.multiple_of(s * H_STRIPE, 8)
        pltpu.sync_copy(h_hbm.at[pl.ds(r0, H_STRIPE)],
                        h_spm.at[pl.ds(r0, H_STRIPE)])

    @pl.when(s == 15)
    def _():
        pltpu.sync_copy(h_hbm.at[pl.ds(15 * H_STRIPE, H_STRIPE_LAST)],
                        h_spm.at[pl.ds(15 * H_STRIPE, H_STRIPE_LAST)])

    pltpu.sync_copy(src_hbm.at[wid], src_v)
    plsc.subcore_barrier()

    def _start_gather(j, buf, sem):
        return pltpu.async_copy(h_spm.at[src_v.at[pl.ds(j * CH, CH)]],
                                buf, sem)

    def _start_write(j, buf, sem):
        return pltpu.async_copy(
            buf, msg_hbm.at[wid, pl.ds(pl.multiple_of(j * CH, CH), CH)], sem)

    def _wait(buf, sem):
        pltpu.make_async_copy(h_hbm.at[pl.ds(0, CH)], buf, sem).wait()

    _start_gather(0, rows0_v, gsem0)
    _start_gather(1, rows1_v, gsem1)

    def body(i, carry):
        j = pl.multiple_of(i * 2, 2)
        _wait(rows0_v, gsem0)
        _start_write(j, rows0_v, wsem0)
        _wait(rows1_v, gsem1)
        _start_write(j + 1, rows1_v, wsem1)

        @pl.when(j + 2 < CHUNKS)
        def _():
            _wait(rows0_v, wsem0)
            _start_gather(j + 2, rows0_v, gsem0)

        @pl.when(j + 3 < CHUNKS)
        def _():
            _wait(rows1_v, wsem1)
            _start_gather(j + 3, rows1_v, gsem1)

        return carry

    lax.fori_loop(0, CHUNKS // 2, body, 0, unroll=False)
    _wait(rows0_v, gsem0)
    _start_write(CHUNKS - 1, rows0_v, wsem0)
    _wait(rows0_v, wsem0)
    _wait(rows1_v, wsem1)


@functools.partial(
    pl.kernel,
    out_type=jax.ShapeDtypeStruct((NC, NP, D), jnp.float32),
    mesh=_sc_mesh,
    scratch_types=[
        pltpu.VMEM((6, SCH), jnp.int32),
        pltpu.VMEM((SCH, D), jnp.float32),
        pltpu.VMEM((SCH, D), jnp.float32),
        pltpu.VMEM((SCH, D), jnp.float32),
        pltpu.VMEM((SCH, D), jnp.float32),
        pltpu.VMEM_SHARED((NP, D), jnp.float32),
        pltpu.SemaphoreType.DMA,
        pltpu.SemaphoreType.DMA,
        pltpu.SemaphoreType.DMA,
        pltpu.SemaphoreType.DMA,
        pltpu.SemaphoreType.DMA,
        pltpu.SemaphoreType.DMA,
        pltpu.SemaphoreType.DMA,
        pltpu.SemaphoreType.DMA,
        pltpu.SemaphoreType.DMA,
        pltpu.SemaphoreType.DMA,
        pltpu.SemaphoreType.DMA,
    ],
)
def _sc_scatter(msg_hbm, dst_hbm, zeros_hbm, out_hbm,
                ring_v, rows0_v, rows1_v, rows2_v, rows3_v, acc,
                rsem0, rsem1, rsem2, rsem3,
                ssem0, ssem1, ssem2, ssem3,
                isem0, isem1, isem2):
    c = lax.axis_index("c")
    s = lax.axis_index("s")
    wid = s * NC + c
    r0 = pl.multiple_of(s * ROWS_PER_TILE, ROWS_PER_TILE)
    pltpu.sync_copy(zeros_hbm.at[pl.ds(r0, ROWS_PER_TILE)],
                    acc.at[pl.ds(r0, ROWS_PER_TILE)])
    plsc.subcore_barrier()

    rows = (rows0_v, rows1_v, rows2_v, rows3_v)
    rsem = (rsem0, rsem1, rsem2, rsem3)
    ssem = (ssem0, ssem1, ssem2, ssem3)
    isem = (isem0, isem1, isem2)

    def _start_read(j, b, sem):
        return pltpu.async_copy(
            msg_hbm.at[wid, pl.ds(pl.multiple_of(j * SCH, SCH), SCH)],
            rows[b], sem)

    def _start_idx(j, slot, sem):
        return pltpu.async_copy(dst_hbm.at[wid, j], ring_v.at[slot], sem)

    def _wait_rows(b, sem):
        pltpu.make_async_copy(msg_hbm.at[0, pl.ds(0, SCH)], rows[b],
                              sem).wait()

    def _wait_idx(slot, sem):
        pltpu.make_async_copy(dst_hbm.at[0, 0], ring_v.at[slot], sem).wait()

    def _chunk(j, b, i3, slot6, first12):
        _wait_rows(b, rsem[b])
        _wait_idx(slot6, isem[i3])
        pltpu.async_copy(rows[b], acc.at[ring_v.at[slot6]], ssem[b],
                         add=True)
        nb = (b + 2) % 4
        if first12:
            if isinstance(j, int) and j < 2:
                _start_read(j + 2, nb, rsem[nb])
            else:
                _wait_rows(nb, ssem[nb])
                _start_read(j + 2, nb, rsem[nb])
        else:
            _wait_rows(nb, ssem[nb])
            _start_read(j + 2, nb, rsem[nb])
        _start_idx(j + 3, (slot6 + 3) % 6, isem[i3])

    _start_read(0, 0, rsem[0])
    _start_read(1, 1, rsem[1])
    _start_idx(0, 0, isem[0])
    _start_idx(1, 1, isem[1])
    _start_idx(2, 2, isem[2])
    for j in range(12):
        _chunk(j, j % 4, j % 3, j % 6, True)

    def body(i, carry):
        j0 = pl.multiple_of(i * 12, 12)
        for k in range(12):
            _chunk(j0 + k, k % 4, k % 3, k % 6, False)
        return carry

    lax.fori_loop(1, 8, body, 0, unroll=False)
    for j in range(96, 108):
        b, i3, slot6 = j % 4, j % 3, j % 6
        _wait_rows(b, rsem[b])
        _wait_idx(slot6, isem[i3])
        pltpu.async_copy(rows[b], acc.at[ring_v.at[slot6]], ssem[b],
                         add=True)
        nb = (b + 2) % 4
        if j + 2 < 108:
            _wait_rows(nb, ssem[nb])
            _start_read(j + 2, nb, rsem[nb])
        if j + 3 < 108:
            _start_idx(j + 3, (slot6 + 3) % 6, isem[i3])
    _wait_rows(0, ssem[0])
    _wait_rows(1, ssem[1])
    _wait_rows(2, ssem[2])
    _wait_rows(3, ssem[3])
    plsc.subcore_barrier()
    pltpu.sync_copy(acc.at[pl.ds(r0, ROWS_PER_TILE)],
                    out_hbm.at[c, pl.ds(r0, ROWS_PER_TILE)])


def _sc_segment_sum(h, src, dst, zeros):
    msg = _sc_materialize(h, src)
    return _sc_scatter(msg, dst, zeros)


def _tc_linear_body(x_ref, w_ref, b_ref, o_ref):
    o_ref[...] = (jnp.dot(x_ref[...], w_ref[...],
                          preferred_element_type=jnp.float32) + b_ref[...])


def _tc_linear(x, W, b):
    return pl.pallas_call(
        _tc_linear_body,
        out_shape=jax.ShapeDtypeStruct((N, H), jnp.float32),
    )(x, W, b.reshape(1, H))


def _tc_tail_body(relu_out, h_ref, p_ref, wa_ref, ba_ref, g_ref, be_ref,
                  wb_ref, bb_ref, o_ref):
    out = h_ref[...] + p_ref[0, :N] + p_ref[1, :N]
    t = (jnp.dot(out, wa_ref[...], preferred_element_type=jnp.float32)
         + ba_ref[...])
    m = jnp.mean(t, axis=0, keepdims=True)
    v = jnp.mean((t - m) ** 2, axis=0, keepdims=True)
    t = (t - m) * lax.rsqrt(v + 1e-5) * g_ref[...] + be_ref[...]
    t = jnp.maximum(t, 0.0)
    l = (jnp.dot(t, wb_ref[...], preferred_element_type=jnp.float32)
         + bb_ref[...])
    cm = jnp.mean(l, axis=0, keepdims=True)
    rn = jnp.sqrt(1e-6 + jnp.sum(l * l, axis=1, keepdims=True))
    res = 20.0 * l / rn - cm
    if relu_out:
        res = jnp.maximum(res, 0.0)
    o_ref[...] = res


def _tc_tail(relu_out, h, p, Wa, ba, g, be, Wb, bb):
    return pl.pallas_call(
        functools.partial(_tc_tail_body, relu_out),
        out_shape=jax.ShapeDtypeStruct((N, H), jnp.float32),
    )(h, p, Wa, ba.reshape(1, H), g.reshape(1, H), be.reshape(1, H),
      Wb, bb.reshape(1, H))


def kernel(x, edge_index, W0, b0, W1a, b1a, g1, be1, W1b, b1b,
           W2a, b2a, g2, be2, W2b, b2b):
    pad = E_PER_W_PAD - E_PER_W
    src = jnp.pad(edge_index[0].reshape(NW, E_PER_W), ((0, 0), (0, pad)),
                  constant_values=0)
    dst = jnp.pad(edge_index[1].reshape(NW, E_PER_W), ((0, 0), (0, pad)),
                  constant_values=N).reshape(NW, SCHUNKS, SCH)
    zeros = jnp.zeros((NP, D), jnp.float32)

    h0 = _tc_linear(x, W0, b0)
    p1 = _sc_segment_sum(h0, src, dst, zeros)
    l1 = _tc_tail(True, h0, p1, W1a, b1a, g1, be1, W1b, b1b)
    p2 = _sc_segment_sum(l1, src, dst, zeros)
    l2 = _tc_tail(False, l1, p2, W2a, b2a, g2, be2, W2b, b2b)
    return l2
